# baseline (speedup 1.0000x reference)
import jax
import jax.numpy as jnp
from jax import lax
from jax.experimental import pallas as pl
from jax.experimental.pallas import tpu as pltpu
from jax.experimental.pallas import tpu_sc as plsc

N_TOKENS = 32768
N_EXPERTS = 64
TOPK = 8
NC, NS, L = 2, 16, 16
NW = NC * NS
TPW = N_TOKENS // NW
PAIRS = TPW // 2
OROWS = TPW * TOPK // 128
XROWS = TPW * N_EXPERTS // 128
PITCH = 129

_GATHER_DNUMS = lax.GatherDimensionNumbers(
    offset_dims=(), collapsed_slice_dims=(0,), start_index_map=(0,))


def _permute(x, idx):
  return lax.gather(x, idx[:, None], _GATHER_DNUMS, slice_sizes=(1,),
                    mode=lax.GatherScatterMode.PROMISE_IN_BOUNDS)


def _topk_body(x_hbm, p_hbm, i_hbm, x_s0, x_s1, x_p, p_v, i_v, sem0, sem1):
  wid = lax.axis_index("s") * NC + lax.axis_index("c")

  lane = lax.iota(jnp.int32, L)
  sel8 = lane < TOPK
  rev8 = jnp.where(sel8, (TOPK - 1) - lane, 0)
  shl8 = jnp.where(sel8, 0, lane - TOPK)
  lane_hi = lane // TOPK
  lane_lo = lane - TOPK * lane_hi
  zero16 = lane * 0
  rowpat = PITCH * (64 * lane_hi + lane_lo)

  bufs = ((x_s0, sem0), (x_s1, sem1))

  def _start(e_hi):
    buf, sem = bufs[e_hi % 2]
    return pltpu.async_copy(
        x_hbm.at[pl.ds((e_hi * 256 + TOPK * wid) * TOPK, 64), :], buf, sem)

  cp = _start(0)
  for e_hi in range(TOPK):
    cp.wait()
    if e_hi + 1 < TOPK:
      cp = _start(e_hi + 1)
    x_s = bufs[e_hi % 2][0]

    @plsc.parallel_loop(0, 64, unroll=8)
    def relayout(r):
      dst = (e_hi * 64 + r) * PITCH
      for k in range(128 // L):
        x_p[pl.ds(dst + k * L, L)] = x_s[r, pl.ds(k * L, L)]

  def token_topk(pbase, col):
    colv = rowpat + (pbase + col)
    ks, vs = [], []
    for c in range(N_EXPERTS // L):
      x = plsc.load_gather(x_p, [colv + PITCH * 128 * c])
      sk, sv = plsc.sort_key_val(x, lane + c * L, descending=True)
      ks.append(sk)
      vs.append(sv)
    k01, v01 = merge8(ks[0], vs[0], ks[1], vs[1])
    k23, v23 = merge8(ks[2], vs[2], ks[3], vs[3])
    ck = jnp.where(sel8, k01, _permute(k23, shl8))
    cv = jnp.where(sel8, v01, _permute(v23, shl8))
    fk, fv = plsc.sort_key_val(ck, cv, descending=True)
    m = _permute(fk, zero16)
    e = jnp.where(sel8, jnp.exp(fk - m), 0.0)
    s = e + _permute(e, lane ^ 1)
    s = s + _permute(s, lane ^ 2)
    s = s + _permute(s, lane ^ 4)
    return e / s, fv

  def merge8(ka, va, kb, vb):
    kr = _permute(kb, rev8)
    vr = _permute(vb, rev8)
    take_a = ka >= kr
    return jnp.where(take_a, ka, kr), jnp.where(take_a, va, vr)

  @plsc.parallel_loop(0, PAIRS, unroll=4)
  def pair_body(t2):
    tcl = t2 // 64
    c0 = (t2 % 64) * 2
    pbase = PITCH * TOPK * tcl
    p_a, v_a = token_topk(pbase, c0)
    p_b, v_b = token_topk(pbase, c0 + 1)
    pp = jnp.where(sel8, p_a, _permute(p_b, shl8))
    vv = jnp.where(sel8, v_a, _permute(v_b, shl8))
    orow = tcl * TOPK + lane_lo
    ocol = c0 + lane_hi
    plsc.store_scatter(p_v, [orow, ocol], pp)
    plsc.store_scatter(i_v, [orow, ocol], vv)

  pltpu.sync_copy(p_v, p_hbm.at[pl.ds(wid * OROWS, OROWS), :])
  pltpu.sync_copy(i_v, i_hbm.at[pl.ds(wid * OROWS, OROWS), :])


_topk_call = pl.kernel(
    _topk_body,
    out_type=(
        jax.ShapeDtypeStruct((N_TOKENS * TOPK // 128, 128), jnp.float32),
        jax.ShapeDtypeStruct((N_TOKENS * TOPK // 128, 128), jnp.int32),
    ),
    mesh=plsc.VectorSubcoreMesh(
        core_axis_name="c", subcore_axis_name="s",
        num_cores=NC, num_subcores=NS),
    scratch_types=[
        pltpu.VMEM((64, 128), jnp.float32),
        pltpu.VMEM((64, 128), jnp.float32),
        pltpu.VMEM((XROWS * PITCH,), jnp.float32),
        pltpu.VMEM((OROWS, 128), jnp.float32),
        pltpu.VMEM((OROWS, 128), jnp.int32),
        pltpu.SemaphoreType.DMA,
        pltpu.SemaphoreType.DMA,
    ],
    compiler_params=pltpu.CompilerParams(needs_layout_passes=False),
)


def _to_tile_order(x):
  return x.T.reshape(TOPK, TOPK, 256, 128).transpose(0, 2, 1, 3).reshape(
      N_TOKENS * N_EXPERTS // 128, 128)


def _from_kmajor(o):
  return o.reshape(N_TOKENS // 128, TOPK, 128).transpose(0, 2, 1).reshape(
      N_TOKENS, TOPK)


def kernel(gating_logits):
  n, e = gating_logits.shape
  assert n == N_TOKENS and e == N_EXPERTS
  probs, idx = _topk_call(_to_tile_order(gating_logits))
  return (_from_kmajor(probs), _from_kmajor(idx))


# speedup vs baseline: 1.0119x; 1.0119x over previous
import jax
import jax.numpy as jnp
from jax import lax
from jax.experimental import pallas as pl
from jax.experimental.pallas import tpu as pltpu
from jax.experimental.pallas import tpu_sc as plsc

N_TOKENS = 32768
N_EXPERTS = 64
TOPK = 8
NC, NS, L = 2, 16, 16
NW = NC * NS
TPW = N_TOKENS // NW
PAIRS = TPW // 2
OROWS = TPW * TOPK // 128
XROWS = TPW * N_EXPERTS // 128
PITCH = 129

_GATHER_DNUMS = lax.GatherDimensionNumbers(
    offset_dims=(), collapsed_slice_dims=(0,), start_index_map=(0,))


def _permute(x, idx):
  return lax.gather(x, idx[:, None], _GATHER_DNUMS, slice_sizes=(1,),
                    mode=lax.GatherScatterMode.PROMISE_IN_BOUNDS)


def _topk_body(x_hbm, p_hbm, i_hbm, x_s0, x_s1, x_p, p_v, i_v, sem0, sem1):
  wid = lax.axis_index("s") * NC + lax.axis_index("c")

  lane = lax.iota(jnp.int32, L)
  sel8 = lane < TOPK
  rev8 = jnp.where(sel8, (TOPK - 1) - lane, 0)
  shl8 = jnp.where(sel8, 0, lane - TOPK)
  lane_hi = lane // TOPK
  lane_lo = lane - TOPK * lane_hi
  rowpat = PITCH * (64 * lane_hi + lane_lo)

  bufs = ((x_s0, sem0), (x_s1, sem1))

  def _start(e_hi):
    buf, sem = bufs[e_hi % 2]
    return pltpu.async_copy(
        x_hbm.at[pl.ds((e_hi * 256 + TOPK * wid) * TOPK, 64), :], buf, sem)

  cp = _start(0)
  for e_hi in range(TOPK):
    cp.wait()
    if e_hi + 1 < TOPK:
      cp = _start(e_hi + 1)
    x_s = bufs[e_hi % 2][0]

    @plsc.parallel_loop(0, 64, unroll=8)
    def relayout(r):
      dst = (e_hi * 64 + r) * PITCH
      for k in range(128 // L):
        x_p[pl.ds(dst + k * L, L)] = x_s[r, pl.ds(k * L, L)]

  def token_topk(pbase, col):
    colv = rowpat + (pbase + col)
    ks, vs = [], []
    for c in range(N_EXPERTS // L):
      x = plsc.load_gather(x_p, [colv + PITCH * 128 * c])
      sk, sv = plsc.sort_key_val(x, lane + c * L, descending=True)
      ks.append(sk)
      vs.append(sv)
    k01, v01 = merge8(ks[0], vs[0], ks[1], vs[1])
    k23, v23 = merge8(ks[2], vs[2], ks[3], vs[3])
    ck = jnp.where(sel8, k01, _permute(k23, shl8))
    cv = jnp.where(sel8, v01, _permute(v23, shl8))
    fk, fv = plsc.sort_key_val(ck, cv, descending=True)
    m = jnp.max(fk)
    e = jnp.where(sel8, jnp.exp(fk - m), 0.0)
    return e / jnp.sum(e), fv

  def merge8(ka, va, kb, vb):
    kr = _permute(kb, rev8)
    vr = _permute(vb, rev8)
    take_a = ka >= kr
    return jnp.where(take_a, ka, kr), jnp.where(take_a, va, vr)

  @plsc.parallel_loop(0, PAIRS, unroll=4)
  def pair_body(t2):
    tcl = t2 // 64
    c0 = (t2 % 64) * 2
    pbase = PITCH * TOPK * tcl
    p_a, v_a = token_topk(pbase, c0)
    p_b, v_b = token_topk(pbase, c0 + 1)
    pp = jnp.where(sel8, p_a, _permute(p_b, shl8))
    vv = jnp.where(sel8, v_a, _permute(v_b, shl8))
    orow = tcl * TOPK + lane_lo
    ocol = c0 + lane_hi
    plsc.store_scatter(p_v, [orow, ocol], pp)
    plsc.store_scatter(i_v, [orow, ocol], vv)

  pltpu.sync_copy(p_v, p_hbm.at[pl.ds(wid * OROWS, OROWS), :])
  pltpu.sync_copy(i_v, i_hbm.at[pl.ds(wid * OROWS, OROWS), :])


_topk_call = pl.kernel(
    _topk_body,
    out_type=(
        jax.ShapeDtypeStruct((N_TOKENS * TOPK // 128, 128), jnp.float32),
        jax.ShapeDtypeStruct((N_TOKENS * TOPK // 128, 128), jnp.int32),
    ),
    mesh=plsc.VectorSubcoreMesh(
        core_axis_name="c", subcore_axis_name="s",
        num_cores=NC, num_subcores=NS),
    scratch_types=[
        pltpu.VMEM((64, 128), jnp.float32),
        pltpu.VMEM((64, 128), jnp.float32),
        pltpu.VMEM((XROWS * PITCH,), jnp.float32),
        pltpu.VMEM((OROWS, 128), jnp.float32),
        pltpu.VMEM((OROWS, 128), jnp.int32),
        pltpu.SemaphoreType.DMA,
        pltpu.SemaphoreType.DMA,
    ],
    compiler_params=pltpu.CompilerParams(needs_layout_passes=False),
)


def _to_tile_order(x):
  return x.T.reshape(TOPK, TOPK, 256, 128).transpose(0, 2, 1, 3).reshape(
      N_TOKENS * N_EXPERTS // 128, 128)


def _from_kmajor(o):
  return o.reshape(N_TOKENS // 128, TOPK, 128).transpose(0, 2, 1).reshape(
      N_TOKENS, TOPK)


def kernel(gating_logits):
  n, e = gating_logits.shape
  assert n == N_TOKENS and e == N_EXPERTS
  probs, idx = _topk_call(_to_tile_order(gating_logits))
  return (_from_kmajor(probs), _from_kmajor(idx))


# speedup vs baseline: 1.0260x; 1.0139x over previous
import jax
import jax.numpy as jnp
from jax import lax
from jax.experimental import pallas as pl
from jax.experimental.pallas import tpu as pltpu
from jax.experimental.pallas import tpu_sc as plsc

N_TOKENS = 32768
N_EXPERTS = 64
TOPK = 8
NC, NS, L = 2, 16, 16
NW = NC * NS
TPW = N_TOKENS // NW
PAIRS = TPW // 2
OROWS = TPW * TOPK // 128
XROWS = TPW * N_EXPERTS // 128
PITCH = 129
OPITCH = 130

_GATHER_DNUMS = lax.GatherDimensionNumbers(
    offset_dims=(), collapsed_slice_dims=(0,), start_index_map=(0,))


def _permute(x, idx):
  return lax.gather(x, idx[:, None], _GATHER_DNUMS, slice_sizes=(1,),
                    mode=lax.GatherScatterMode.PROMISE_IN_BOUNDS)


def _topk_body(x_hbm, p_hbm, i_hbm, x_s0, x_s1, x_p, p_p, i_p, p_v, i_v,
               sem0, sem1):
  wid = lax.axis_index("s") * NC + lax.axis_index("c")

  lane = lax.iota(jnp.int32, L)
  sel8 = lane < TOPK
  rev8 = jnp.where(sel8, (TOPK - 1) - lane, 0)
  shl8 = jnp.where(sel8, 0, lane - TOPK)
  lane_hi = lane // TOPK
  lane_lo = lane - TOPK * lane_hi
  rowpat = PITCH * (64 * lane_hi + lane_lo)

  bufs = ((x_s0, sem0), (x_s1, sem1))

  def _start(e_hi):
    buf, sem = bufs[e_hi % 2]
    return pltpu.async_copy(
        x_hbm.at[pl.ds((e_hi * 256 + TOPK * wid) * TOPK, 64), :], buf, sem)

  cp = _start(0)
  for e_hi in range(TOPK):
    cp.wait()
    if e_hi + 1 < TOPK:
      cp = _start(e_hi + 1)
    x_s = bufs[e_hi % 2][0]

    @plsc.parallel_loop(0, 64, unroll=8)
    def relayout(r):
      dst = (e_hi * 64 + r) * PITCH
      for k in range(128 // L):
        x_p[pl.ds(dst + k * L, L)] = x_s[r, pl.ds(k * L, L)]

  def token_topk(pbase, col):
    colv = rowpat + (pbase + col)
    ks, vs = [], []
    for c in range(N_EXPERTS // L):
      x = plsc.load_gather(x_p, [colv + PITCH * 128 * c])
      sk, sv = plsc.sort_key_val(x, lane + c * L, descending=True)
      ks.append(sk)
      vs.append(sv)
    k01, v01 = merge8(ks[0], vs[0], ks[1], vs[1])
    k23, v23 = merge8(ks[2], vs[2], ks[3], vs[3])
    ck = jnp.where(sel8, k01, _permute(k23, shl8))
    cv = jnp.where(sel8, v01, _permute(v23, shl8))
    fk, fv = plsc.sort_key_val(ck, cv, descending=True)
    m = jnp.max(fk)
    e = jnp.where(sel8, jnp.exp(fk - m), 0.0)
    return e / jnp.sum(e), fv

  def merge8(ka, va, kb, vb):
    kr = _permute(kb, rev8)
    vr = _permute(vb, rev8)
    take_a = ka >= kr
    return jnp.where(take_a, ka, kr), jnp.where(take_a, va, vr)

  @plsc.parallel_loop(0, PAIRS, unroll=4)
  def pair_body(t2):
    tcl = t2 // 64
    c0 = (t2 % 64) * 2
    pbase = PITCH * TOPK * tcl
    p_a, v_a = token_topk(pbase, c0)
    p_b, v_b = token_topk(pbase, c0 + 1)
    pp = jnp.where(sel8, p_a, _permute(p_b, shl8))
    vv = jnp.where(sel8, v_a, _permute(v_b, shl8))
    oidx = OPITCH * (tcl * TOPK + lane_lo) + (c0 + lane_hi)
    plsc.store_scatter(p_p, [oidx], pp)
    plsc.store_scatter(i_p, [oidx], vv)

  @plsc.parallel_loop(0, OROWS, unroll=8)
  def depitch(r):
    for k in range(128 // L):
      p_v[r, pl.ds(k * L, L)] = p_p[pl.ds(OPITCH * r + k * L, L)]
      i_v[r, pl.ds(k * L, L)] = i_p[pl.ds(OPITCH * r + k * L, L)]

  pltpu.sync_copy(p_v, p_hbm.at[pl.ds(wid * OROWS, OROWS), :])
  pltpu.sync_copy(i_v, i_hbm.at[pl.ds(wid * OROWS, OROWS), :])


_topk_call = pl.kernel(
    _topk_body,
    out_type=(
        jax.ShapeDtypeStruct((N_TOKENS * TOPK // 128, 128), jnp.float32),
        jax.ShapeDtypeStruct((N_TOKENS * TOPK // 128, 128), jnp.int32),
    ),
    mesh=plsc.VectorSubcoreMesh(
        core_axis_name="c", subcore_axis_name="s",
        num_cores=NC, num_subcores=NS),
    scratch_types=[
        pltpu.VMEM((64, 128), jnp.float32),
        pltpu.VMEM((64, 128), jnp.float32),
        pltpu.VMEM((XROWS * PITCH,), jnp.float32),
        pltpu.VMEM((OROWS * OPITCH,), jnp.float32),
        pltpu.VMEM((OROWS * OPITCH,), jnp.int32),
        pltpu.VMEM((OROWS, 128), jnp.float32),
        pltpu.VMEM((OROWS, 128), jnp.int32),
        pltpu.SemaphoreType.DMA,
        pltpu.SemaphoreType.DMA,
    ],
    compiler_params=pltpu.CompilerParams(needs_layout_passes=False),
)


def _to_tile_order(x):
  return x.T.reshape(TOPK, TOPK, 256, 128).transpose(0, 2, 1, 3).reshape(
      N_TOKENS * N_EXPERTS // 128, 128)


def _from_kmajor(o):
  return o.reshape(N_TOKENS // 128, TOPK, 128).transpose(0, 2, 1).reshape(
      N_TOKENS, TOPK)


def kernel(gating_logits):
  n, e = gating_logits.shape
  assert n == N_TOKENS and e == N_EXPERTS
  probs, idx = _topk_call(_to_tile_order(gating_logits))
  return (_from_kmajor(probs), _from_kmajor(idx))
